# baseline (speedup 1.0000x reference)
import jax
import jax.numpy as jnp
from jax import lax
from jax.experimental import pallas as pl
from jax.experimental.pallas import tpu as pltpu
from jax.experimental.pallas import tpu_sc as plsc

_GRID = 64
_S = 32
_P = 8192
_VDIM = 3
_G3 = _GRID * _GRID * _GRID
_GW = _G3 * _VDIM
_NT = 16
_RPC = _S // 2
_PT = _P // _NT
_HLF = _PT // 2
_NG = 24
_HW = _HLF * _NG
_SL = _GW // _NT

_OFFJ = tuple(
    (((j // 6) >> 1) * _GRID * _GRID + ((j // 6) & 1) * _GRID) * _VDIM
    + (j % 6)
    for j in range(_NG))


def _tc_idx_body(p_ref, x_ref, idx_ref, w_ref):
    p = p_ref[0]
    c = p[:, 0:1]
    off = p[:, 1:2]
    sp = p[:, 2:3]
    rad = p[0:1, 3:4]
    d = x_ref[0] - c
    q = d[0:1] * d[0:1] + d[1:2] * d[1:2] + d[2:3] * d[2:3]
    nd = jnp.maximum(jnp.sqrt(q) / rad, 1.0)
    t = (d / nd + off) / sp
    it = t.astype(jnp.int32)
    w_ref[0] = t - it.astype(jnp.float32)
    rowe = ((it[0:1] * _GRID + it[1:2]) * _GRID + it[2:3]) * _VDIM
    for j in range(_NG):
        idx_ref[0, j:j + 1, :] = rowe + _OFFJ[j]


def _tc_blend_body(cv_ref, w_ref, o_ref):
    wx = w_ref[0, 0:1, :]
    wy = w_ref[0, 1:2, :]
    wz = w_ref[0, 2:3, :]
    ex = 1.0 - wx
    ey = 1.0 - wy
    ez = 1.0 - wz
    for c in range(_VDIM):
        acc = None
        for qp in range(4):
            for dz in range(2):
                wk = ((wx if (qp >> 1) else ex)
                      * (wy if (qp & 1) else ey) * (wz if dz else ez))
                v = wk * cv_ref[0, qp * 6 + 3 * dz + c:qp * 6 + 3 * dz + c + 1, :]
                acc = v if acc is None else acc + v
        o_ref[0, c:c + 1, :] = acc


def _sc_body(idx_hbm, tab_hbm, cv_hbm,
             idx_v, cv_v, spm_a, spm_b, gsem, ssem):
    core = lax.axis_index("c")
    sid = lax.axis_index("s")
    shbase = core * _RPC

    def stage_issue(r_next, spm):
        src = tab_hbm.at[pl.ds((shbase + r_next) * _GW + sid * _SL, _SL)]
        pltpu.async_copy(src, spm.at[pl.ds(sid * _SL, _SL)], ssem)

    def stage_drain(r_next, spm):
        src = tab_hbm.at[pl.ds((shbase + r_next) * _GW + sid * _SL, _SL)]
        pltpu.make_async_copy(
            src, spm.at[pl.ds(sid * _SL, _SL)], ssem).wait()

    def round_body(r, spm, spm_next):
        sh = shbase + r
        lbase = (sh * _P + sid * _PT) * _NG

        @pl.when(r + 1 < _RPC)
        def _():
            stage_issue(r + 1, spm_next)

        def half(h, carry):
            hoff = lbase + h * _HW
            pltpu.sync_copy(idx_hbm.at[pl.ds(hoff, _HW)], idx_v)
            pltpu.async_copy(spm.at[idx_v.at[...]], cv_v, gsem).wait()
            pltpu.sync_copy(cv_v, cv_hbm.at[pl.ds(hoff, _HW)])
            return carry

        lax.fori_loop(0, 2, half, 0)

        @pl.when(r + 1 < _RPC)
        def _():
            stage_drain(r + 1, spm_next)

        plsc.subcore_barrier()

    stage_issue(0, spm_a)
    stage_drain(0, spm_a)
    plsc.subcore_barrier()

    def superround(sr, carry):
        round_body(2 * sr, spm_a, spm_b)
        round_body(2 * sr + 1, spm_b, spm_a)
        return carry

    lax.fori_loop(0, _RPC // 2, superround, 0)


def kernel(x, s, values, center, radius, bounding_box, spacing):
    si = s.astype(jnp.int32)
    c = center[si]
    r = radius[si]
    bb0 = bounding_box[si, 0]
    sp = spacing[si]
    params = jnp.concatenate(
        [jnp.stack([c, c - bb0, sp, jnp.broadcast_to(r[:, None], (_S, 3))],
                   axis=2),
         jnp.zeros((_S, 3, 12), jnp.float32)], axis=2)
    xt = x.transpose(0, 2, 1)
    idxp, w = pl.pallas_call(
        _tc_idx_body,
        grid=(_S,),
        in_specs=[pl.BlockSpec((1, 3, 16), lambda i: (i, 0, 0)),
                  pl.BlockSpec((1, 3, _P), lambda i: (i, 0, 0))],
        out_specs=[pl.BlockSpec((1, _NG, _P), lambda i: (i, 0, 0)),
                   pl.BlockSpec((1, 3, _P), lambda i: (i, 0, 0))],
        out_shape=[jax.ShapeDtypeStruct((_S, _NG, _P), jnp.int32),
                   jax.ShapeDtypeStruct((_S, 3, _P), jnp.float32)],
    )(params, xt)
    idxflat = idxp.transpose(0, 2, 1).reshape(_S * _P * _NG)
    fn = pl.kernel(
        _sc_body,
        out_type=jax.ShapeDtypeStruct((_S * _P * _NG,), jnp.float32),
        mesh=plsc.VectorSubcoreMesh(core_axis_name="c", subcore_axis_name="s"),
        scratch_types=[
            pltpu.VMEM((_HW,), jnp.int32),
            pltpu.VMEM((_HW,), jnp.float32),
            pltpu.VMEM_SHARED((_GW,), jnp.float32),
            pltpu.VMEM_SHARED((_GW,), jnp.float32),
            pltpu.SemaphoreType.DMA,
            pltpu.SemaphoreType.DMA,
        ],
    )
    cvraw = fn(idxflat, values.reshape(_S * _GW))
    cvt = cvraw.reshape(_S, _P, _NG).transpose(0, 2, 1)
    out = pl.pallas_call(
        _tc_blend_body,
        grid=(_S,),
        in_specs=[pl.BlockSpec((1, _NG, _P), lambda i: (i, 0, 0)),
                  pl.BlockSpec((1, 3, _P), lambda i: (i, 0, 0))],
        out_specs=pl.BlockSpec((1, 3, _P), lambda i: (i, 0, 0)),
        out_shape=jax.ShapeDtypeStruct((_S, 3, _P), jnp.float32),
    )(cvt, w)
    return out.transpose(0, 2, 1)
